# baseline (speedup 1.0000x reference)
import functools

import jax
import jax.numpy as jnp
from jax import lax
from jax.experimental import pallas as pl
from jax.experimental.pallas import tpu as pltpu
from jax.experimental.pallas import tpu_sc as plsc

IN_FEATURES = 512
NUM_CODEBOOKS = 2
NUM_CODEWORDS = 320
CW_PAD = 384
CODEWORD_DIM = 128
ROWS = 4 * 1024
BLK = 512
GRID = ROWS // BLK

NC, NS = 2, 16
NW = NC * NS
N_IDX = ROWS * NUM_CODEBOOKS
CHUNK = N_IDX // NW


def _tc_body(x_ref, w_ref, b_ref, ids_ref, perp_ref, counts_ref):
    pid = pl.program_id(0)

    @pl.when(pid == 0)
    def _init():
        counts_ref[...] = jnp.zeros_like(counts_ref)

    logits = (
        jnp.dot(x_ref[...], w_ref[...], preferred_element_type=jnp.float32)
        + b_ref[...]
    )
    iota = lax.broadcasted_iota(jnp.int32, (BLK, CW_PAD), 1)
    big = jnp.int32(2**30)
    for n in range(NUM_CODEBOOKS):
        l = logits[:, n * CW_PAD : (n + 1) * CW_PAD]
        m = jnp.max(l, axis=1, keepdims=True)
        cand = jnp.where(l == m, iota, big)
        idx = jnp.min(cand, axis=1, keepdims=True)
        onehot = (iota == idx).astype(jnp.float32)
        counts_ref[n : n + 1, :] += jnp.sum(onehot, axis=0, keepdims=True)
        ids_ref[:, n : n + 1] = idx + n * NUM_CODEWORDS

    @pl.when(pid == GRID - 1)
    def _fin():
        p = counts_ref[...] * (1.0 / ROWS)
        ent = jnp.sum(p * jnp.log(p + 1e-7), axis=1, keepdims=True)
        perp_ref[...] = jnp.broadcast_to(jnp.sum(jnp.exp(-ent)), (1, 1))


def _tc_stage(x2d, w_pad, b_pad):
    return pl.pallas_call(
        _tc_body,
        grid=(GRID,),
        in_specs=[
            pl.BlockSpec((BLK, IN_FEATURES), lambda i: (i, 0)),
            pl.BlockSpec((IN_FEATURES, NUM_CODEBOOKS * CW_PAD), lambda i: (0, 0)),
            pl.BlockSpec((1, NUM_CODEBOOKS * CW_PAD), lambda i: (0, 0)),
        ],
        out_specs=[
            pl.BlockSpec((BLK, NUM_CODEBOOKS), lambda i: (i, 0)),
            pl.BlockSpec((1, 1), lambda i: (0, 0)),
        ],
        out_shape=[
            jax.ShapeDtypeStruct((ROWS, NUM_CODEBOOKS), jnp.int32),
            jax.ShapeDtypeStruct((1, 1), jnp.float32),
        ],
        scratch_shapes=[pltpu.VMEM((NUM_CODEBOOKS, CW_PAD), jnp.float32)],
    )(x2d, w_pad, b_pad)


@functools.lru_cache(maxsize=1)
def _make_sc_gather():
    @functools.partial(
        pl.kernel,
        mesh=plsc.VectorSubcoreMesh(core_axis_name="c", subcore_axis_name="s"),
        out_type=jax.ShapeDtypeStruct((N_IDX, CODEWORD_DIM), jnp.float32),
        scratch_types=[
            pltpu.VMEM((CHUNK // 128, 128), jnp.int32),
            pltpu.VMEM((CHUNK, CODEWORD_DIM), jnp.float32),
            pltpu.SemaphoreType.DMA,
        ],
    )
    def _sc_gather(table_hbm, idx_hbm, out_hbm, idx_v, rows_v, sem):
        wid = lax.axis_index("s") * NC + lax.axis_index("c")
        n_sub = CHUNK // 128
        pltpu.sync_copy(idx_hbm.at[pl.ds(wid * n_sub, n_sub)], idx_v)
        copies = []
        for j in range(n_sub):
            copies.append(
                pltpu.async_copy(
                    table_hbm.at[idx_v.at[j]],
                    rows_v.at[pl.ds(j * 128, 128)],
                    sem,
                )
            )
        for c in copies:
            c.wait()
        pltpu.sync_copy(rows_v, out_hbm.at[pl.ds(wid * CHUNK, CHUNK)])

    return _sc_gather


def kernel(x, codebooks, W, b):
    bsz, nf, _ = x.shape
    x2d = x.reshape(bsz * nf, IN_FEATURES)
    pad_w = jnp.zeros((IN_FEATURES, CW_PAD - NUM_CODEWORDS), W.dtype)
    w_pad = jnp.concatenate(
        [W[:, :NUM_CODEWORDS], pad_w, W[:, NUM_CODEWORDS:], pad_w], axis=1
    )
    pad_b = jnp.full((CW_PAD - NUM_CODEWORDS,), -1e30, b.dtype)
    b_pad = jnp.concatenate(
        [b[:NUM_CODEWORDS], pad_b, b[NUM_CODEWORDS:], pad_b]
    ).reshape(1, NUM_CODEBOOKS * CW_PAD)

    ids, perp = _tc_stage(x2d, w_pad, b_pad)

    table = codebooks.reshape(NUM_CODEBOOKS * NUM_CODEWORDS, CODEWORD_DIM)
    idx = ids.reshape(NW * (CHUNK // 128), 128)
    rows = _make_sc_gather()(table, idx)

    quantized = rows.reshape(bsz, nf, NUM_CODEBOOKS * CODEWORD_DIM)
    return quantized, perp.reshape(())
